# baseline (speedup 1.0000x reference)
import functools

import jax
import jax.numpy as jnp
from jax import lax
from jax.experimental import pallas as pl
from jax.experimental.pallas import tpu as pltpu
from jax.experimental.pallas import tpu_sc as plsc

N_NODES = 10000
N_EDGES = 320000
F = 128

NC = 2
NS = 16
NW = NC * NS
K = 48
NCHUNK = 210
PAD_EDGES = NW * NCHUNK * K
ACC_ROWS = 10016
ZA = 624
ZTAIL = ACC_ROWS - NS * ZA
WA = 624
WTAIL = N_NODES - NS * WA
NBUF = 5
NGROUP = NCHUNK // NBUF

_sc_mesh = plsc.VectorSubcoreMesh(
    core_axis_name="c", subcore_axis_name="s", num_cores=NC, num_subcores=NS)


def _sc_body(g_hbm, src_hbm, dst_hbm, zero_hbm, out_hbm,
             src_v, dst_v, rows_a, sem_ga, sem_sa, acc):
    c = lax.axis_index("c")
    s = lax.axis_index("s")
    w = c * NS + s
    pltpu.sync_copy(zero_hbm.at[pl.ds(s * ZA, ZA)], acc.at[pl.ds(s * ZA, ZA)])

    @pl.when(s == NS - 1)
    def _zero_tail():
        pltpu.sync_copy(zero_hbm.at[pl.ds(NS * ZA, ZTAIL)],
                        acc.at[pl.ds(NS * ZA, ZTAIL)])

    pltpu.sync_copy(src_hbm.at[w], src_v)
    pltpu.sync_copy(dst_hbm.at[w], dst_v)
    plsc.subcore_barrier()

    for b in range(NBUF):
        pltpu.async_copy(g_hbm.at[src_v.at[b]], rows_a.at[b], sem_ga.at[b])

    def group(g, carry):
        j0 = g * NBUF
        for b in range(NBUF):
            pltpu.make_async_copy(
                g_hbm.at[src_v.at[j0 + b]], rows_a.at[b], sem_ga.at[b]).wait()
            pltpu.async_copy(
                rows_a.at[b], acc.at[dst_v.at[j0 + b]], sem_sa.at[b],
                add=True)
        for b in range(NBUF):
            pltpu.make_async_copy(
                rows_a.at[b], acc.at[dst_v.at[j0 + b]], sem_sa.at[b]).wait()
            jn = jnp.minimum(j0 + NBUF + b, NCHUNK - 1)

            @pl.when(g < NGROUP - 1)
            def _next_gather():
                pltpu.async_copy(g_hbm.at[src_v.at[jn]], rows_a.at[b],
                                 sem_ga.at[b])

        return carry

    lax.fori_loop(0, NGROUP, group, 0)
    plsc.subcore_barrier()
    pltpu.sync_copy(acc.at[pl.ds(s * WA, WA)], out_hbm.at[c, pl.ds(s * WA, WA)])

    @pl.when(s == NS - 1)
    def _write_tail():
        pltpu.sync_copy(acc.at[pl.ds(NS * WA, WTAIL)],
                        out_hbm.at[c, pl.ds(NS * WA, WTAIL)])


_sc_scatter = functools.partial(
    pl.kernel,
    out_type=jax.ShapeDtypeStruct((NC, N_NODES, F), jnp.float32),
    mesh=_sc_mesh,
    scratch_types=[
        pltpu.VMEM((NCHUNK, K), jnp.int32),
        pltpu.VMEM((NCHUNK, K), jnp.int32),
        pltpu.VMEM((NBUF, K, F), jnp.float32),
        pltpu.SemaphoreType.DMA((NBUF,)),
        pltpu.SemaphoreType.DMA((NBUF,)),
        pltpu.VMEM_SHARED((ACC_ROWS, F), jnp.float32),
    ],
    compiler_params=pltpu.CompilerParams(use_tc_tiling_on_sc=False),
)(_sc_body)


ROWS_B = 1000


def _mm0_body(px_ref, nx_ref, w_ref, op_ref, on_ref):
    op_ref[...] = jnp.dot(px_ref[...], w_ref[0],
                          preferred_element_type=jnp.float32)
    on_ref[...] = jnp.dot(nx_ref[...], w_ref[1],
                          preferred_element_type=jnp.float32)


def _mid_body(pp_ref, np_ref, w_ref, op_ref, on_ref):
    ap = jnp.maximum(pp_ref[0] + pp_ref[1], 0.0)
    an = jnp.maximum(np_ref[0] + np_ref[1], 0.0)
    op_ref[...] = jnp.dot(ap, w_ref[0], preferred_element_type=jnp.float32)
    on_ref[...] = jnp.dot(an, w_ref[1], preferred_element_type=jnp.float32)


def _final_body(px_ref, pp_ref, nx_ref, np_ref, o_ref):
    o_ref[:, 0:F] = px_ref[...]
    o_ref[:, F:2 * F] = jnp.maximum(pp_ref[0] + pp_ref[1], 0.0)
    o_ref[:, 2 * F:3 * F] = nx_ref[...]
    o_ref[:, 3 * F:4 * F] = jnp.maximum(np_ref[0] + np_ref[1], 0.0)


_GRID = N_NODES // ROWS_B
_x_spec = pl.BlockSpec((ROWS_B, F), lambda i: (i, 0))
_w_spec = pl.BlockSpec((2, F, F), lambda i: (0, 0, 0))
_p_spec = pl.BlockSpec((NC, ROWS_B, F), lambda i: (0, i, 0))
_o_spec = pl.BlockSpec((ROWS_B, F), lambda i: (i, 0))
_o_type = jax.ShapeDtypeStruct((N_NODES, F), jnp.float32)
_f_spec = pl.BlockSpec((ROWS_B, 4 * F), lambda i: (i, 0))
_f_type = jax.ShapeDtypeStruct((N_NODES, 4 * F), jnp.float32)

_mm0 = pl.pallas_call(_mm0_body, grid=(_GRID,),
                      in_specs=[_x_spec, _x_spec, _w_spec],
                      out_specs=[_o_spec, _o_spec],
                      out_shape=[_o_type, _o_type])
_mid = pl.pallas_call(_mid_body, grid=(_GRID,),
                      in_specs=[_p_spec, _p_spec, _w_spec],
                      out_specs=[_o_spec, _o_spec],
                      out_shape=[_o_type, _o_type])
_final = pl.pallas_call(_final_body, grid=(_GRID,),
                        in_specs=[_x_spec, _p_spec, _x_spec, _p_spec],
                        out_specs=_f_spec, out_shape=_f_type)


def _prep_edges(edge_index):
    pad = PAD_EDGES - N_EDGES
    pad_src = (jnp.arange(pad, dtype=jnp.int32) * 37) % N_NODES
    pad_dst = N_NODES + (jnp.arange(pad, dtype=jnp.int32) % (ACC_ROWS - N_NODES))
    src = jnp.concatenate(
        [edge_index[0], pad_src]).reshape(NW, NCHUNK, K)
    dst = jnp.concatenate(
        [edge_index[1], pad_dst]).reshape(NW, NCHUNK, K)
    return src, dst


def kernel(pos_x, pos_edge_index, neg_x, neg_edge_index,
           pos_W0, pos_W1, pos_W2, neg_W0, neg_W1, neg_W2):
    zeros_hbm = jnp.zeros((ACC_ROWS, F), jnp.float32)
    psrc, pdst = _prep_edges(pos_edge_index)
    nsrc, ndst = _prep_edges(neg_edge_index)
    w0 = jnp.stack([pos_W0, neg_W0])
    w1 = jnp.stack([pos_W1, neg_W1])
    w2 = jnp.stack([pos_W2, neg_W2])
    gp, gn = _mm0(pos_x, neg_x, w0)
    pp = _sc_scatter(gp, psrc, pdst, zeros_hbm)
    pn = _sc_scatter(gn, nsrc, ndst, zeros_hbm)
    gp, gn = _mid(pp, pn, w1)
    pp = _sc_scatter(gp, psrc, pdst, zeros_hbm)
    pn = _sc_scatter(gn, nsrc, ndst, zeros_hbm)
    gp, gn = _mid(pp, pn, w2)
    pp = _sc_scatter(gp, psrc, pdst, zeros_hbm)
    pn = _sc_scatter(gn, nsrc, ndst, zeros_hbm)
    return _final(pos_x, pp, neg_x, pn)


# speedup vs baseline: 1.0356x; 1.0356x over previous
import functools

import jax
import jax.numpy as jnp
from jax import lax
from jax.experimental import pallas as pl
from jax.experimental.pallas import tpu as pltpu
from jax.experimental.pallas import tpu_sc as plsc

N_NODES = 10000
N_EDGES = 320000
F = 128

NC = 2
NS = 16
NW = NC * NS
K = 48
NCHUNK = 210
PAD_EDGES = NW * NCHUNK * K
ACC_ROWS = 10016
ZA = 624
ZTAIL = ACC_ROWS - NS * ZA
WA = 624
WTAIL = N_NODES - NS * WA
NBUF = 5
NGROUP = NCHUNK // NBUF

_sc_mesh = plsc.VectorSubcoreMesh(
    core_axis_name="c", subcore_axis_name="s", num_cores=NC, num_subcores=NS)


def _sc_body(g_hbm, src_hbm, dst_hbm, zero_hbm, out_hbm,
             src_v, dst_v, rows_a, sem_ga, sem_sa, acc):
    c = lax.axis_index("c")
    s = lax.axis_index("s")
    w = c * NS + s
    pltpu.sync_copy(zero_hbm.at[pl.ds(s * ZA, ZA)], acc.at[pl.ds(s * ZA, ZA)])

    @pl.when(s == NS - 1)
    def _zero_tail():
        pltpu.sync_copy(zero_hbm.at[pl.ds(NS * ZA, ZTAIL)],
                        acc.at[pl.ds(NS * ZA, ZTAIL)])

    pltpu.sync_copy(src_hbm.at[w], src_v)
    pltpu.sync_copy(dst_hbm.at[w], dst_v)
    plsc.subcore_barrier()

    for b in range(NBUF):
        pltpu.async_copy(g_hbm.at[src_v.at[b]], rows_a.at[b], sem_ga.at[b])

    def group(g, carry):
        j0 = g * NBUF
        for b in range(NBUF):
            pltpu.make_async_copy(
                g_hbm.at[src_v.at[j0 + b]], rows_a.at[b], sem_ga.at[b]).wait()
            pltpu.async_copy(
                rows_a.at[b], acc.at[dst_v.at[j0 + b]], sem_sa.at[b],
                add=True)
        for b in range(NBUF):
            pltpu.make_async_copy(
                rows_a.at[b], acc.at[dst_v.at[j0 + b]], sem_sa.at[b]).wait()
            jn = jnp.minimum(j0 + NBUF + b, NCHUNK - 1)

            @pl.when(g < NGROUP - 1)
            def _next_gather():
                pltpu.async_copy(g_hbm.at[src_v.at[jn]], rows_a.at[b],
                                 sem_ga.at[b])

        return carry

    lax.fori_loop(0, NGROUP, group, 0)
    plsc.subcore_barrier()
    pltpu.sync_copy(acc.at[pl.ds(s * WA, WA)], out_hbm.at[c, pl.ds(s * WA, WA)])

    @pl.when(s == NS - 1)
    def _write_tail():
        pltpu.sync_copy(acc.at[pl.ds(NS * WA, WTAIL)],
                        out_hbm.at[c, pl.ds(NS * WA, WTAIL)])


_sc_scatter = functools.partial(
    pl.kernel,
    out_type=jax.ShapeDtypeStruct((NC, N_NODES, F), jnp.float32),
    mesh=_sc_mesh,
    scratch_types=[
        pltpu.VMEM((NCHUNK, K), jnp.int32),
        pltpu.VMEM((NCHUNK, K), jnp.int32),
        pltpu.VMEM((NBUF, K, F), jnp.float32),
        pltpu.SemaphoreType.DMA((NBUF,)),
        pltpu.SemaphoreType.DMA((NBUF,)),
        pltpu.VMEM_SHARED((ACC_ROWS, F), jnp.float32),
    ],
    compiler_params=pltpu.CompilerParams(use_tc_tiling_on_sc=False),
)(_sc_body)


ROWS_B = 1000


def _mm0_body(x_ref, w_ref, o_ref):
    o_ref[...] = jnp.dot(x_ref[...], w_ref[...],
                         preferred_element_type=jnp.float32)


def _mid_body(p_ref, w_ref, o_ref):
    a = jnp.maximum(p_ref[0] + p_ref[1], 0.0)
    o_ref[...] = jnp.dot(a, w_ref[...], preferred_element_type=jnp.float32)


def _final_body(px_ref, pp_ref, nx_ref, np_ref, o_ref):
    o_ref[:, 0:F] = px_ref[...]
    o_ref[:, F:2 * F] = jnp.maximum(pp_ref[0] + pp_ref[1], 0.0)
    o_ref[:, 2 * F:3 * F] = nx_ref[...]
    o_ref[:, 3 * F:4 * F] = jnp.maximum(np_ref[0] + np_ref[1], 0.0)


_GRID = N_NODES // ROWS_B
_x_spec = pl.BlockSpec((ROWS_B, F), lambda i: (i, 0))
_w_spec = pl.BlockSpec((F, F), lambda i: (0, 0))
_p_spec = pl.BlockSpec((NC, ROWS_B, F), lambda i: (0, i, 0))
_o_spec = pl.BlockSpec((ROWS_B, F), lambda i: (i, 0))
_o_type = jax.ShapeDtypeStruct((N_NODES, F), jnp.float32)
_f_spec = pl.BlockSpec((ROWS_B, 4 * F), lambda i: (i, 0))
_f_type = jax.ShapeDtypeStruct((N_NODES, 4 * F), jnp.float32)

_mm0 = pl.pallas_call(_mm0_body, grid=(_GRID,), in_specs=[_x_spec, _w_spec],
                      out_specs=_o_spec, out_shape=_o_type)
_mid = pl.pallas_call(_mid_body, grid=(_GRID,), in_specs=[_p_spec, _w_spec],
                      out_specs=_o_spec, out_shape=_o_type)
_final = pl.pallas_call(_final_body, grid=(_GRID,),
                        in_specs=[_x_spec, _p_spec, _x_spec, _p_spec],
                        out_specs=_f_spec, out_shape=_f_type)


def _prep_edges(edge_index):
    pad = PAD_EDGES - N_EDGES
    pad_src = (jnp.arange(pad, dtype=jnp.int32) * 37) % N_NODES
    pad_dst = N_NODES + (jnp.arange(pad, dtype=jnp.int32) % (ACC_ROWS - N_NODES))
    src = jnp.concatenate(
        [edge_index[0], pad_src]).reshape(NW, NCHUNK, K)
    dst = jnp.concatenate(
        [edge_index[1], pad_dst]).reshape(NW, NCHUNK, K)
    return src, dst


def kernel(pos_x, pos_edge_index, neg_x, neg_edge_index,
           pos_W0, pos_W1, pos_W2, neg_W0, neg_W1, neg_W2):
    zeros_hbm = jnp.zeros((ACC_ROWS, F), jnp.float32)
    psrc, pdst = _prep_edges(pos_edge_index)
    nsrc, ndst = _prep_edges(neg_edge_index)
    gp = _mm0(pos_x, pos_W0)
    gn = _mm0(neg_x, neg_W0)
    pp = _sc_scatter(gp, psrc, pdst, zeros_hbm)
    pn = _sc_scatter(gn, nsrc, ndst, zeros_hbm)
    gp = _mid(pp, pos_W1)
    gn = _mid(pn, neg_W1)
    pp = _sc_scatter(gp, psrc, pdst, zeros_hbm)
    pn = _sc_scatter(gn, nsrc, ndst, zeros_hbm)
    gp = _mid(pp, pos_W2)
    gn = _mid(pn, neg_W2)
    pp = _sc_scatter(gp, psrc, pdst, zeros_hbm)
    pn = _sc_scatter(gn, nsrc, ndst, zeros_hbm)
    return _final(pos_x, pp, neg_x, pn)


# speedup vs baseline: 1.0380x; 1.0023x over previous
import functools

import jax
import jax.numpy as jnp
from jax import lax
from jax.experimental import pallas as pl
from jax.experimental.pallas import tpu as pltpu
from jax.experimental.pallas import tpu_sc as plsc

N_NODES = 10000
N_EDGES = 320000
F = 128

NC = 2
NS = 16
NW = NC * NS
K = 40
NCHUNK = 252
PAD_EDGES = NW * NCHUNK * K
ACC_ROWS = 10016
ZA = 624
ZTAIL = ACC_ROWS - NS * ZA
WA = 624
WTAIL = N_NODES - NS * WA
NBUF = 6
NGROUP = NCHUNK // NBUF

_sc_mesh = plsc.VectorSubcoreMesh(
    core_axis_name="c", subcore_axis_name="s", num_cores=NC, num_subcores=NS)


def _sc_body(g_hbm, src_hbm, dst_hbm, zero_hbm, out_hbm,
             src_v, dst_v, rows_a, sem_ga, sem_sa, acc):
    c = lax.axis_index("c")
    s = lax.axis_index("s")
    w = c * NS + s
    pltpu.sync_copy(zero_hbm.at[pl.ds(s * ZA, ZA)], acc.at[pl.ds(s * ZA, ZA)])

    @pl.when(s == NS - 1)
    def _zero_tail():
        pltpu.sync_copy(zero_hbm.at[pl.ds(NS * ZA, ZTAIL)],
                        acc.at[pl.ds(NS * ZA, ZTAIL)])

    pltpu.sync_copy(src_hbm.at[w], src_v)
    pltpu.sync_copy(dst_hbm.at[w], dst_v)
    plsc.subcore_barrier()

    for b in range(NBUF):
        pltpu.async_copy(g_hbm.at[src_v.at[b]], rows_a.at[b], sem_ga.at[b])

    def group(g, carry):
        j0 = g * NBUF
        for b in range(NBUF):
            pltpu.make_async_copy(
                g_hbm.at[src_v.at[j0 + b]], rows_a.at[b], sem_ga.at[b]).wait()
            pltpu.async_copy(
                rows_a.at[b], acc.at[dst_v.at[j0 + b]], sem_sa.at[b],
                add=True)
        for b in range(NBUF):
            pltpu.make_async_copy(
                rows_a.at[b], acc.at[dst_v.at[j0 + b]], sem_sa.at[b]).wait()
            jn = jnp.minimum(j0 + NBUF + b, NCHUNK - 1)

            @pl.when(g < NGROUP - 1)
            def _next_gather():
                pltpu.async_copy(g_hbm.at[src_v.at[jn]], rows_a.at[b],
                                 sem_ga.at[b])

        return carry

    lax.fori_loop(0, NGROUP, group, 0)
    plsc.subcore_barrier()
    pltpu.sync_copy(acc.at[pl.ds(s * WA, WA)], out_hbm.at[c, pl.ds(s * WA, WA)])

    @pl.when(s == NS - 1)
    def _write_tail():
        pltpu.sync_copy(acc.at[pl.ds(NS * WA, WTAIL)],
                        out_hbm.at[c, pl.ds(NS * WA, WTAIL)])


_sc_scatter = functools.partial(
    pl.kernel,
    out_type=jax.ShapeDtypeStruct((NC, N_NODES, F), jnp.float32),
    mesh=_sc_mesh,
    scratch_types=[
        pltpu.VMEM((NCHUNK, K), jnp.int32),
        pltpu.VMEM((NCHUNK, K), jnp.int32),
        pltpu.VMEM((NBUF, K, F), jnp.float32),
        pltpu.SemaphoreType.DMA((NBUF,)),
        pltpu.SemaphoreType.DMA((NBUF,)),
        pltpu.VMEM_SHARED((ACC_ROWS, F), jnp.float32),
    ],
    compiler_params=pltpu.CompilerParams(use_tc_tiling_on_sc=False),
)(_sc_body)


ROWS_B = 1000


def _mm0_body(x_ref, w_ref, o_ref):
    o_ref[...] = jnp.dot(x_ref[...], w_ref[...],
                         preferred_element_type=jnp.float32)


def _mid_body(p_ref, w_ref, o_ref):
    a = jnp.maximum(p_ref[0] + p_ref[1], 0.0)
    o_ref[...] = jnp.dot(a, w_ref[...], preferred_element_type=jnp.float32)


def _final_body(px_ref, pp_ref, nx_ref, np_ref, o_ref):
    o_ref[:, 0:F] = px_ref[...]
    o_ref[:, F:2 * F] = jnp.maximum(pp_ref[0] + pp_ref[1], 0.0)
    o_ref[:, 2 * F:3 * F] = nx_ref[...]
    o_ref[:, 3 * F:4 * F] = jnp.maximum(np_ref[0] + np_ref[1], 0.0)


_GRID = N_NODES // ROWS_B
_x_spec = pl.BlockSpec((ROWS_B, F), lambda i: (i, 0))
_w_spec = pl.BlockSpec((F, F), lambda i: (0, 0))
_p_spec = pl.BlockSpec((NC, ROWS_B, F), lambda i: (0, i, 0))
_o_spec = pl.BlockSpec((ROWS_B, F), lambda i: (i, 0))
_o_type = jax.ShapeDtypeStruct((N_NODES, F), jnp.float32)
_f_spec = pl.BlockSpec((ROWS_B, 4 * F), lambda i: (i, 0))
_f_type = jax.ShapeDtypeStruct((N_NODES, 4 * F), jnp.float32)

_mm0 = pl.pallas_call(_mm0_body, grid=(_GRID,), in_specs=[_x_spec, _w_spec],
                      out_specs=_o_spec, out_shape=_o_type)
_mid = pl.pallas_call(_mid_body, grid=(_GRID,), in_specs=[_p_spec, _w_spec],
                      out_specs=_o_spec, out_shape=_o_type)
_final = pl.pallas_call(_final_body, grid=(_GRID,),
                        in_specs=[_x_spec, _p_spec, _x_spec, _p_spec],
                        out_specs=_f_spec, out_shape=_f_type)


def _prep_edges(edge_index):
    pad = PAD_EDGES - N_EDGES
    pad_src = (jnp.arange(pad, dtype=jnp.int32) * 37) % N_NODES
    pad_dst = N_NODES + (jnp.arange(pad, dtype=jnp.int32) % (ACC_ROWS - N_NODES))
    src = jnp.concatenate(
        [edge_index[0], pad_src]).reshape(NW, NCHUNK, K)
    dst = jnp.concatenate(
        [edge_index[1], pad_dst]).reshape(NW, NCHUNK, K)
    return src, dst


def kernel(pos_x, pos_edge_index, neg_x, neg_edge_index,
           pos_W0, pos_W1, pos_W2, neg_W0, neg_W1, neg_W2):
    zeros_hbm = jnp.zeros((ACC_ROWS, F), jnp.float32)
    psrc, pdst = _prep_edges(pos_edge_index)
    nsrc, ndst = _prep_edges(neg_edge_index)
    gp = _mm0(pos_x, pos_W0)
    gn = _mm0(neg_x, neg_W0)
    pp = _sc_scatter(gp, psrc, pdst, zeros_hbm)
    pn = _sc_scatter(gn, nsrc, ndst, zeros_hbm)
    gp = _mid(pp, pos_W1)
    gn = _mid(pn, neg_W1)
    pp = _sc_scatter(gp, psrc, pdst, zeros_hbm)
    pn = _sc_scatter(gn, nsrc, ndst, zeros_hbm)
    gp = _mid(pp, pos_W2)
    gn = _mid(pn, neg_W2)
    pp = _sc_scatter(gp, psrc, pdst, zeros_hbm)
    pn = _sc_scatter(gn, nsrc, ndst, zeros_hbm)
    return _final(pos_x, pp, neg_x, pn)
